# baseline (speedup 1.0000x reference)
import functools
import jax
import jax.numpy as jnp
from jax import lax
from jax.experimental import pallas as pl
from jax.experimental.pallas import tpu as pltpu
from jax.experimental.pallas import tpu_sc as plsc

DIM = 64
BATCH = 16384
LANES = 16
NBLK = 7813
CPB = 7
CHW = CPB * 128
NCHK = 35
NBUF = 2
IPC = 2048
LCAP = 2048
RING = 8

_info = plsc.get_sparse_core_info()
NC = _info.num_cores
NS = _info.num_subcores
NW = NC * NS
BPW = BATCH // NW
NGRP = BPW // LANES
TPAD = 17

_mesh = plsc.VectorSubcoreMesh(core_axis_name="c", subcore_axis_name="s")
_params = pltpu.CompilerParams(needs_layout_passes=False)

_DNUMS = lax.GatherDimensionNumbers(
    offset_dims=(), collapsed_slice_dims=(0,), start_index_map=(0,))


def _dyn_gather(v, j):
    return lax.gather(v, j[:, None], _DNUMS, slice_sizes=(1,),
                      mode=lax.GatherScatterMode.PROMISE_IN_BOUNDS)


@functools.partial(
    pl.kernel,
    mesh=_mesh,
    out_type=jax.ShapeDtypeStruct((BATCH * DIM,), jnp.float32),
    scratch_types=[
        pltpu.VMEM((IPC,), jnp.int32),
        pltpu.VMEM((LCAP + LANES,), jnp.int32),
        pltpu.VMEM((LCAP + LANES,), jnp.int32),
        pltpu.VMEM((NBUF, DIM, CHW), jnp.float32),
        pltpu.VMEM((RING * DIM,), jnp.float32),
        pltpu.SemaphoreType.DMA,
        pltpu.SemaphoreType.DMA,
    ],
    compiler_params=_params,
)
def _scan_gather(iw_hbm, tT_hbm, rows_hbm,
                 idx_v, bl_v, il_v, st_v, rb_v, sem_in, sem_out):
    wid = lax.axis_index("s") * NC + lax.axis_index("c")
    bs = wid * 244 + jnp.minimum(wid, 5)
    bn = 244 + (wid < 5).astype(jnp.int32)
    be = bs + bn

    lane = lax.iota(jnp.int32, LANES)

    def chunk_col(c):
        cb = jnp.minimum(bs + c * CPB, NBLK - CPB)
        return pl.multiple_of(cb * 128, 128)

    def fire(c, buf):
        pltpu.async_copy(tT_hbm.at[:, pl.ds(chunk_col(c), CHW)],
                         st_v.at[buf], sem_in)

    def drain_in():
        pltpu.make_async_copy(tT_hbm.at[:, pl.ds(0, CHW)], st_v.at[0],
                              sem_in).wait()

    for p in range(NBUF - 1):
        fire(jnp.int32(p), p)

    def piece(p, cnt):
        pltpu.sync_copy(iw_hbm.at[pl.ds(p * IPC, IPC)], idx_v)

        def prescan(g, cnt):
            v = idx_v[pl.ds(g * LANES, LANES)]
            blk = v >> 7
            m = (blk >= bs) & (blk < be)
            bl = lane + (p * IPC + g * LANES)
            plsc.store_compressed(bl_v.at[pl.ds(cnt, LANES)], bl, mask=m)
            plsc.store_compressed(il_v.at[pl.ds(cnt, LANES)], v, mask=m)
            return cnt + plsc.all_reduce_population_count(m)[0]

        return lax.fori_loop(0, IPC // LANES, prescan, cnt)

    lcnt = lax.fori_loop(0, BATCH // IPC, piece, jnp.int32(0))
    nlv = (lcnt + LANES - 1) // LANES

    def chunk_body(c, fired):
        buf = lax.rem(c, NBUF)
        @pl.when(c + NBUF - 1 < NCHK)
        def _():
            fire(c + NBUF - 1, lax.rem(c + NBUF - 1, NBUF))
        drain_in()
        col0 = chunk_col(c)
        lo = col0 >> 7
        hi = lo + CPB

        def list_vreg(j, fired):
            vi = il_v[pl.ds(j * LANES, LANES)]
            vb = bl_v[pl.ds(j * LANES, LANES)]
            valid = (lane + j * LANES) < lcnt
            m0 = ((vi >> 7) >= lo) & ((vi >> 7) < hi) & valid

            def member(k, carry):
                m, fired = carry
                j1 = plsc.all_reduce_ffs(m != 0)
                idx_s = _dyn_gather(vi, j1)[0]
                b_s = _dyn_gather(vb, j1)[0]
                col = jnp.full((LANES,), idx_s - col0, jnp.int32)
                slot = lax.rem(fired, RING)
                @pl.when(fired >= RING)
                def _():
                    pltpu.make_async_copy(
                        rb_v.at[pl.ds(0, DIM)],
                        rows_hbm.at[pl.ds(0, DIM)], sem_out).wait()
                for k4 in range(DIM // LANES):
                    rows = lane + (k4 * LANES)
                    rv = plsc.load_gather(st_v.at[buf], [rows, col])
                    rb_v[pl.ds(slot * DIM + k4 * LANES, LANES)] = rv
                pltpu.async_copy(rb_v.at[pl.ds(slot * DIM, DIM)],
                                 rows_hbm.at[pl.ds(b_s * DIM, DIM)], sem_out)
                m = m & (lane != j1[0]).astype(jnp.int32)
                return m, fired + 1

            n0 = plsc.all_reduce_population_count(m0)[0]
            _, fired = lax.fori_loop(0, n0, member,
                                     (m0.astype(jnp.int32), fired))
            return fired

        return lax.fori_loop(0, nlv, list_vreg, fired)

    fired = lax.fori_loop(0, NCHK, chunk_body, jnp.int32(0))

    def drain_out(i, carry):
        @pl.when(i < jnp.minimum(fired, RING))
        def _():
            pltpu.make_async_copy(rb_v.at[pl.ds(0, DIM)],
                                  rows_hbm.at[pl.ds(0, DIM)], sem_out).wait()
        return carry

    lax.fori_loop(0, RING, drain_out, 0)


@functools.partial(
    pl.kernel,
    mesh=_mesh,
    out_type=jax.ShapeDtypeStruct((BATCH,), jnp.float32),
    scratch_types=[
        pltpu.VMEM((BPW * DIM,), jnp.float32),
        pltpu.VMEM((BPW * DIM,), jnp.float32),
        pltpu.VMEM((BPW,), jnp.float32),
        pltpu.VMEM((LANES * TPAD,), jnp.float32),
    ],
    compiler_params=_params,
)
def _dot_phase(crows_hbm, xrows_hbm, out_hbm, cr_v, xr_v, o_v, tb_v):
    wid = lax.axis_index("s") * NC + lax.axis_index("c")
    base = wid * BPW
    pltpu.sync_copy(crows_hbm.at[pl.ds(base * DIM, BPW * DIM)], cr_v)
    pltpu.sync_copy(xrows_hbm.at[pl.ds(base * DIM, BPW * DIM)], xr_v)
    lane = lax.iota(jnp.int32, LANES)

    def group(g, carry):
        w0 = g * LANES * DIM
        for j in range(LANES):
            w = w0 + j * DIM
            acc = cr_v[pl.ds(w, LANES)] * xr_v[pl.ds(w, LANES)]
            for k in range(1, DIM // LANES):
                acc = acc + (cr_v[pl.ds(w + k * LANES, LANES)]
                             * xr_v[pl.ds(w + k * LANES, LANES)])
            tb_v[pl.ds(j * TPAD, LANES)] = acc
        tot = plsc.load_gather(tb_v, [lane * TPAD])
        for i in range(1, LANES):
            tot = tot + plsc.load_gather(tb_v, [lane * TPAD + i])
        o_v[pl.ds(g * LANES, LANES)] = tot
        return carry

    lax.fori_loop(0, NGRP, group, 0)
    pltpu.sync_copy(o_v, out_hbm.at[pl.ds(base, BPW)])


def kernel(center_word, context_word, center_table, context_table):
    cw = center_word.astype(jnp.int32)
    xw = context_word.astype(jnp.int32)
    crows = _scan_gather(cw, jnp.transpose(center_table))
    xrows = _scan_gather(xw, jnp.transpose(context_table))
    return _dot_phase(crows, xrows)


# speedup vs baseline: 1.0282x; 1.0282x over previous
import functools
import jax
import jax.numpy as jnp
from jax import lax
from jax.experimental import pallas as pl
from jax.experimental.pallas import tpu as pltpu
from jax.experimental.pallas import tpu_sc as plsc

DIM = 64
BATCH = 16384
LANES = 16
NBLK = 7813
CPB = 5
CHW = CPB * 128
NCHK = 49
NBUF = 3
IPC = 2048
LCAP = 2048
RING = 8

_info = plsc.get_sparse_core_info()
NC = _info.num_cores
NS = _info.num_subcores
NW = NC * NS
BPW = BATCH // NW
NGRP = BPW // LANES
TPAD = 17

_mesh = plsc.VectorSubcoreMesh(core_axis_name="c", subcore_axis_name="s")
_params = pltpu.CompilerParams(needs_layout_passes=False)

_DNUMS = lax.GatherDimensionNumbers(
    offset_dims=(), collapsed_slice_dims=(0,), start_index_map=(0,))


def _dyn_gather(v, j):
    return lax.gather(v, j[:, None], _DNUMS, slice_sizes=(1,),
                      mode=lax.GatherScatterMode.PROMISE_IN_BOUNDS)


@functools.partial(
    pl.kernel,
    mesh=_mesh,
    out_type=jax.ShapeDtypeStruct((BATCH * DIM,), jnp.float32),
    scratch_types=[
        pltpu.VMEM((IPC,), jnp.int32),
        pltpu.VMEM((LCAP + LANES,), jnp.int32),
        pltpu.VMEM((LCAP + LANES,), jnp.int32),
        pltpu.VMEM((NBUF, DIM, CHW), jnp.float32),
        pltpu.VMEM((RING * DIM,), jnp.float32),
        pltpu.SemaphoreType.DMA,
        pltpu.SemaphoreType.DMA,
    ],
    compiler_params=_params,
)
def _scan_gather(iw_hbm, tT_hbm, rows_hbm,
                 idx_v, bl_v, il_v, st_v, rb_v, sem_in, sem_out):
    wid = lax.axis_index("s") * NC + lax.axis_index("c")
    bs = wid * 244 + jnp.minimum(wid, 5)
    bn = 244 + (wid < 5).astype(jnp.int32)
    be = bs + bn

    lane = lax.iota(jnp.int32, LANES)

    def chunk_col(c):
        cb = jnp.minimum(bs + c * CPB, NBLK - CPB)
        return pl.multiple_of(cb * 128, 128)

    def fire(c, buf):
        pltpu.async_copy(tT_hbm.at[:, pl.ds(chunk_col(c), CHW)],
                         st_v.at[buf], sem_in)

    def drain_in():
        pltpu.make_async_copy(tT_hbm.at[:, pl.ds(0, CHW)], st_v.at[0],
                              sem_in).wait()

    for p in range(NBUF - 1):
        fire(jnp.int32(p), p)

    def piece(p, cnt):
        pltpu.sync_copy(iw_hbm.at[pl.ds(p * IPC, IPC)], idx_v)

        def prescan(g, cnt):
            v = idx_v[pl.ds(g * LANES, LANES)]
            blk = v >> 7
            m = (blk >= bs) & (blk < be)
            bl = lane + (p * IPC + g * LANES)
            plsc.store_compressed(bl_v.at[pl.ds(cnt, LANES)], bl, mask=m)
            plsc.store_compressed(il_v.at[pl.ds(cnt, LANES)], v, mask=m)
            return cnt + plsc.all_reduce_population_count(m)[0]

        return lax.fori_loop(0, IPC // LANES, prescan, cnt)

    lcnt = lax.fori_loop(0, BATCH // IPC, piece, jnp.int32(0))
    nlv = (lcnt + LANES - 1) // LANES

    def chunk_body(c, fired):
        buf = lax.rem(c, NBUF)
        @pl.when(c + NBUF - 1 < NCHK)
        def _():
            fire(c + NBUF - 1, lax.rem(c + NBUF - 1, NBUF))
        drain_in()
        col0 = chunk_col(c)
        lo = col0 >> 7
        hi = lo + CPB

        def list_vreg(j, fired):
            vi = il_v[pl.ds(j * LANES, LANES)]
            vb = bl_v[pl.ds(j * LANES, LANES)]
            valid = (lane + j * LANES) < lcnt
            m0 = ((vi >> 7) >= lo) & ((vi >> 7) < hi) & valid

            def member(k, carry):
                m, fired = carry
                j1 = plsc.all_reduce_ffs(m != 0)
                idx_s = _dyn_gather(vi, j1)[0]
                b_s = _dyn_gather(vb, j1)[0]
                col = jnp.full((LANES,), idx_s - col0, jnp.int32)
                slot = lax.rem(fired, RING)
                @pl.when(fired >= RING)
                def _():
                    pltpu.make_async_copy(
                        rb_v.at[pl.ds(0, DIM)],
                        rows_hbm.at[pl.ds(0, DIM)], sem_out).wait()
                for k4 in range(DIM // LANES):
                    rows = lane + (k4 * LANES)
                    rv = plsc.load_gather(st_v.at[buf], [rows, col])
                    rb_v[pl.ds(slot * DIM + k4 * LANES, LANES)] = rv
                pltpu.async_copy(rb_v.at[pl.ds(slot * DIM, DIM)],
                                 rows_hbm.at[pl.ds(b_s * DIM, DIM)], sem_out)
                m = m & (lane != j1[0]).astype(jnp.int32)
                return m, fired + 1

            n0 = plsc.all_reduce_population_count(m0)[0]
            _, fired = lax.fori_loop(0, n0, member,
                                     (m0.astype(jnp.int32), fired))
            return fired

        return lax.fori_loop(0, nlv, list_vreg, fired)

    fired = lax.fori_loop(0, NCHK, chunk_body, jnp.int32(0))

    def drain_out(i, carry):
        @pl.when(i < jnp.minimum(fired, RING))
        def _():
            pltpu.make_async_copy(rb_v.at[pl.ds(0, DIM)],
                                  rows_hbm.at[pl.ds(0, DIM)], sem_out).wait()
        return carry

    lax.fori_loop(0, RING, drain_out, 0)


@functools.partial(
    pl.kernel,
    mesh=_mesh,
    out_type=jax.ShapeDtypeStruct((BATCH,), jnp.float32),
    scratch_types=[
        pltpu.VMEM((BPW * DIM,), jnp.float32),
        pltpu.VMEM((BPW * DIM,), jnp.float32),
        pltpu.VMEM((BPW,), jnp.float32),
        pltpu.VMEM((LANES * TPAD,), jnp.float32),
        pltpu.SemaphoreType.DMA,
    ],
    compiler_params=_params,
)
def _dot_phase(crows_hbm, xrows_hbm, out_hbm, cr_v, xr_v, o_v, tb_v, sem):
    wid = lax.axis_index("s") * NC + lax.axis_index("c")
    base = wid * BPW
    c1 = pltpu.async_copy(crows_hbm.at[pl.ds(base * DIM, BPW * DIM)], cr_v, sem)
    c2 = pltpu.async_copy(xrows_hbm.at[pl.ds(base * DIM, BPW * DIM)], xr_v, sem)
    c1.wait()
    c2.wait()
    lane = lax.iota(jnp.int32, LANES)

    def group(g, carry):
        w0 = g * LANES * DIM
        for j in range(LANES):
            w = w0 + j * DIM
            acc = cr_v[pl.ds(w, LANES)] * xr_v[pl.ds(w, LANES)]
            for k in range(1, DIM // LANES):
                acc = acc + (cr_v[pl.ds(w + k * LANES, LANES)]
                             * xr_v[pl.ds(w + k * LANES, LANES)])
            tb_v[pl.ds(j * TPAD, LANES)] = acc
        tot = plsc.load_gather(tb_v, [lane * TPAD])
        for i in range(1, LANES):
            tot = tot + plsc.load_gather(tb_v, [lane * TPAD + i])
        o_v[pl.ds(g * LANES, LANES)] = tot
        return carry

    lax.fori_loop(0, NGRP, group, 0)
    pltpu.sync_copy(o_v, out_hbm.at[pl.ds(base, BPW)])


def kernel(center_word, context_word, center_table, context_table):
    cw = center_word.astype(jnp.int32)
    xw = context_word.astype(jnp.int32)
    crows = _scan_gather(cw, jnp.transpose(center_table))
    xrows = _scan_gather(xw, jnp.transpose(context_table))
    return _dot_phase(crows, xrows)
